# baseline (speedup 1.0000x reference)
import functools

import jax
import jax.numpy as jnp
from jax import lax
from jax.experimental import pallas as pl
from jax.experimental.pallas import tpu as pltpu
from jax.experimental.pallas import tpu_sc as plsc

N = 10000
E = 320000
IN_DIM = 128
HID = 16
OUT2 = 8

NC = 2
NS = 16
BLK = 128
EB = 2560
EPAD = EB * BLK
B0 = 52
B1 = 108
NPAD = 10240
RPT = NPAD // NS
NBUF = 4


def _tile_rows(c, s):
    row0 = jnp.where(c == 0, s * B0, NS * B0 + s * B1)
    return row0


@functools.cache
def _make_sc_degree():
  kern = functools.partial(
      pl.kernel,
      out_type=jax.ShapeDtypeStruct((NC, NPAD), jnp.float32),
      mesh=plsc.VectorSubcoreMesh(core_axis_name="c", subcore_axis_name="s"),
      scratch_types=[
          pltpu.VMEM((B1, BLK), jnp.int32),
          pltpu.VMEM((BLK,), jnp.float32),
          pltpu.VMEM((BLK,), jnp.float32),
          pltpu.VMEM_SHARED((NPAD,), jnp.float32),
          pltpu.SemaphoreType.DMA,
      ],
      compiler_params=pltpu.CompilerParams(use_tc_tiling_on_sc=False),
  )
  return kern(_sc_degree_body)


def _sc_degree_body(dst_hbm, out_hbm, dstv, ones_v, zbuf, acc, sem):
    c = lax.axis_index("c")
    s = lax.axis_index("s")
    row0 = _tile_rows(c, s)

    def fill(i, _):
        zbuf[pl.ds(i * 16, 16)] = jnp.zeros((16,), jnp.float32)
        ones_v[pl.ds(i * 16, 16)] = jnp.ones((16,), jnp.float32)
        return 0

    lax.fori_loop(0, BLK // 16, fill, 0)

    def zero(i, _):
        pltpu.sync_copy(zbuf, acc.at[pl.ds(s * RPT + i * BLK, BLK)])
        return 0

    lax.fori_loop(0, RPT // BLK, zero, 0)
    plsc.subcore_barrier()

    def run(nblk):
        pltpu.sync_copy(dst_hbm.at[pl.ds(row0, nblk)],
                        dstv.at[pl.ds(0, nblk)])

        def blk(j, _):
            pltpu.async_copy(ones_v, acc.at[dstv.at[j]], sem, add=True)
            return 0

        lax.fori_loop(0, nblk, blk, 0)

        def drain(j, _):
            pltpu.make_async_copy(ones_v, acc.at[dstv.at[j]], sem).wait()
            return 0

        lax.fori_loop(0, nblk, drain, 0)

    @pl.when(c == 0)
    def _():
        run(B0)

    @pl.when(c == 1)
    def _():
        run(B1)

    plsc.subcore_barrier()
    pltpu.sync_copy(acc.at[pl.ds(s * RPT, RPT)],
                    out_hbm.at[c, pl.ds(s * RPT, RPT)])


@functools.cache
def _make_sc_aggregate(width):
  kern = functools.partial(
      pl.kernel,
      out_type=jax.ShapeDtypeStruct((NC, NPAD, width), jnp.float32),
      mesh=plsc.VectorSubcoreMesh(core_axis_name="c", subcore_axis_name="s"),
      scratch_types=(
          [pltpu.VMEM((B1, BLK), jnp.int32),
           pltpu.VMEM((B1, BLK), jnp.int32)]
          + [pltpu.VMEM((BLK, width), jnp.float32) for _ in range(NBUF)]
          + [pltpu.VMEM_SHARED((NPAD, width), jnp.float32)]
          + [pltpu.SemaphoreType.DMA for _ in range(NBUF)]
      ),
      compiler_params=pltpu.CompilerParams(use_tc_tiling_on_sc=False),
  )
  return kern(functools.partial(_sc_aggregate_body, width))


def _sc_aggregate_body(width, g_hbm, src_hbm, dst_hbm, zeros_hbm, out_hbm,
                       srcv, dstv, *rest):
    bufs = rest[:NBUF]
    acc = rest[NBUF]
    sems = rest[NBUF + 1:]
    c = lax.axis_index("c")
    s = lax.axis_index("s")
    row0 = _tile_rows(c, s)

    pltpu.sync_copy(zeros_hbm, acc.at[pl.ds(s * RPT, RPT)])
    plsc.subcore_barrier()

    def run(nblk):
        pltpu.sync_copy(src_hbm.at[pl.ds(row0, nblk)],
                        srcv.at[pl.ds(0, nblk)])
        pltpu.sync_copy(dst_hbm.at[pl.ds(row0, nblk)],
                        dstv.at[pl.ds(0, nblk)])

        def ring(i, _):
            for k in range(NBUF):
                j = i * NBUF + k

                @pl.when(i > 0)
                def _():
                    pltpu.make_async_copy(bufs[k],
                                          acc.at[dstv.at[j - NBUF]],
                                          sems[k]).wait()
                pltpu.async_copy(g_hbm.at[srcv.at[j]], bufs[k], sems[k])
            for k in range(NBUF):
                j = i * NBUF + k
                pltpu.make_async_copy(g_hbm.at[srcv.at[j]], bufs[k],
                                      sems[k]).wait()
                pltpu.async_copy(bufs[k], acc.at[dstv.at[j]], sems[k],
                                 add=True)
            return 0

        lax.fori_loop(0, nblk // NBUF, ring, 0)
        for k in range(NBUF):
            pltpu.make_async_copy(bufs[k], acc.at[dstv.at[nblk - NBUF + k]],
                                  sems[k]).wait()

    @pl.when(c == 0)
    def _():
        run(B0)

    @pl.when(c == 1)
    def _():
        run(B1)

    plsc.subcore_barrier()
    pltpu.sync_copy(acc.at[pl.ds(s * RPT, RPT)],
                    out_hbm.at[c, pl.ds(s * RPT, RPT)])


def _tc_mm_prep_body(x_ref, w_ref, deg_ref, g_ref, dinv_ref):
    h = jnp.dot(x_ref[...], w_ref[...], preferred_element_type=jnp.float32)
    deg = deg_ref[0] + deg_ref[1] + 1.0
    dinv = lax.rsqrt(deg)
    dinv_ref[...] = dinv
    g_ref[...] = h * dinv


def _tc_layer2_body(q_ref, g1_ref, dinv_ref, b1_ref, w2_ref, g2_ref):
    dinv = dinv_ref[...]
    z = (q_ref[0] + q_ref[1] + g1_ref[...]) * dinv + b1_ref[...]
    z = jnp.maximum(z, 0.0)
    h2 = jnp.dot(z, w2_ref[...], preferred_element_type=jnp.float32)
    g2_ref[...] = h2 * dinv


def _tc_final_body(r_ref, g2_ref, dinv_ref, b2_ref, o_ref):
    logits = (r_ref[0] + r_ref[1] + g2_ref[...]) * dinv_ref[...] + b2_ref[...]
    mask = lax.broadcasted_iota(jnp.int32, logits.shape, 1) < 7
    lm = jnp.where(mask, logits, -jnp.inf)
    mx = jnp.max(lm, axis=1, keepdims=True)
    ex = jnp.where(mask, jnp.exp(logits - mx), 0.0)
    lse = jnp.log(jnp.sum(ex, axis=1, keepdims=True))
    o_ref[...] = logits - mx - lse


_ROWS = 1024
_GRID = NPAD // _ROWS

_tc_mm_prep = pl.pallas_call(
    _tc_mm_prep_body,
    grid=(_GRID,),
    in_specs=[pl.BlockSpec((_ROWS, IN_DIM), lambda i: (i, 0)),
              pl.BlockSpec((IN_DIM, HID), lambda i: (0, 0)),
              pl.BlockSpec((NC, _ROWS, 1), lambda i: (0, i, 0))],
    out_specs=[pl.BlockSpec((_ROWS, HID), lambda i: (i, 0)),
               pl.BlockSpec((_ROWS, 1), lambda i: (i, 0))],
    out_shape=[jax.ShapeDtypeStruct((NPAD, HID), jnp.float32),
               jax.ShapeDtypeStruct((NPAD, 1), jnp.float32)],
)

_tc_layer2 = pl.pallas_call(
    _tc_layer2_body,
    grid=(_GRID,),
    in_specs=[pl.BlockSpec((NC, _ROWS, HID), lambda i: (0, i, 0)),
              pl.BlockSpec((_ROWS, HID), lambda i: (i, 0)),
              pl.BlockSpec((_ROWS, 1), lambda i: (i, 0)),
              pl.BlockSpec((1, HID), lambda i: (0, 0)),
              pl.BlockSpec((HID, OUT2), lambda i: (0, 0))],
    out_specs=pl.BlockSpec((_ROWS, OUT2), lambda i: (i, 0)),
    out_shape=jax.ShapeDtypeStruct((NPAD, OUT2), jnp.float32),
)

_tc_final = pl.pallas_call(
    _tc_final_body,
    grid=(_GRID,),
    in_specs=[pl.BlockSpec((NC, _ROWS, OUT2), lambda i: (0, i, 0)),
              pl.BlockSpec((_ROWS, OUT2), lambda i: (i, 0)),
              pl.BlockSpec((_ROWS, 1), lambda i: (i, 0)),
              pl.BlockSpec((1, OUT2), lambda i: (0, 0))],
    out_specs=pl.BlockSpec((_ROWS, OUT2), lambda i: (i, 0)),
    out_shape=jax.ShapeDtypeStruct((NPAD, OUT2), jnp.float32),
)


@jax.jit
def kernel(x, edge_index, W1, b1, W2, b2):
    ei = edge_index.astype(jnp.int32)
    pad = jnp.full((EPAD - E,), N, jnp.int32)
    src2 = jnp.concatenate([ei[0], pad]).reshape(EB, BLK)
    dst2 = jnp.concatenate([ei[1], pad]).reshape(EB, BLK)
    xp = jnp.pad(x, ((0, NPAD - N), (0, 0)))
    w2p = jnp.zeros((HID, OUT2), jnp.float32).at[:, :7].set(W2)
    b1r = b1.reshape(1, HID)
    b2r = jnp.zeros((1, OUT2), jnp.float32).at[0, :7].set(b2)
    z16 = jnp.zeros((RPT, HID), jnp.float32)
    z8 = jnp.zeros((RPT, OUT2), jnp.float32)

    deg_parts = _make_sc_degree()(dst2).reshape(NC, NPAD, 1)
    g1, dinv = _tc_mm_prep(xp, W1, deg_parts)
    q = _make_sc_aggregate(HID)(g1, src2, dst2, z16)
    g2 = _tc_layer2(q, g1, dinv, b1r, w2p)
    r = _make_sc_aggregate(OUT2)(g2, src2, dst2, z8)
    out = _tc_final(r, g2, dinv, b2r)
    return out[:N, :7]


# speedup vs baseline: 1.1198x; 1.1198x over previous
import functools

import jax
import jax.numpy as jnp
from jax import lax
from jax.experimental import pallas as pl
from jax.experimental.pallas import tpu as pltpu
from jax.experimental.pallas import tpu_sc as plsc

N = 10000
E = 320000
IN_DIM = 128
HID = 16
OUT2 = 8

NC = 2
NS = 16
BLK = 128
EB = 2560
EPAD = EB * BLK
B0 = 108
B1 = 52
BMAX = max(B0, B1)
NPAD = 10240
RPT = NPAD // NS
NBUF = 4


def _tile_rows(c, s):
    row0 = jnp.where(c == 0, s * B0, NS * B0 + s * B1)
    return row0


@functools.cache
def _make_sc_degree():
  kern = functools.partial(
      pl.kernel,
      out_type=jax.ShapeDtypeStruct((NC, NPAD), jnp.float32),
      mesh=plsc.VectorSubcoreMesh(core_axis_name="c", subcore_axis_name="s"),
      scratch_types=[
          pltpu.VMEM((BMAX, BLK), jnp.int32),
          pltpu.VMEM((BLK,), jnp.float32),
          pltpu.VMEM((BLK,), jnp.float32),
          pltpu.VMEM_SHARED((NPAD,), jnp.float32),
          pltpu.SemaphoreType.DMA,
      ],
      compiler_params=pltpu.CompilerParams(use_tc_tiling_on_sc=False),
  )
  return kern(_sc_degree_body)


def _sc_degree_body(dst_hbm, out_hbm, dstv, ones_v, zbuf, acc, sem):
    c = lax.axis_index("c")
    s = lax.axis_index("s")
    row0 = _tile_rows(c, s)

    def fill(i, _):
        zbuf[pl.ds(i * 16, 16)] = jnp.zeros((16,), jnp.float32)
        ones_v[pl.ds(i * 16, 16)] = jnp.ones((16,), jnp.float32)
        return 0

    lax.fori_loop(0, BLK // 16, fill, 0)

    def zero(i, _):
        pltpu.sync_copy(zbuf, acc.at[pl.ds(s * RPT + i * BLK, BLK)])
        return 0

    lax.fori_loop(0, RPT // BLK, zero, 0)
    plsc.subcore_barrier()

    def run(nblk):
        pltpu.sync_copy(dst_hbm.at[pl.ds(row0, nblk)],
                        dstv.at[pl.ds(0, nblk)])

        def blk(j, _):
            pltpu.async_copy(ones_v, acc.at[dstv.at[j]], sem, add=True)
            return 0

        lax.fori_loop(0, nblk, blk, 0)

        def drain(j, _):
            pltpu.make_async_copy(ones_v, acc.at[dstv.at[j]], sem).wait()
            return 0

        lax.fori_loop(0, nblk, drain, 0)

    @pl.when(c == 0)
    def _():
        run(B0)

    @pl.when(c == 1)
    def _():
        run(B1)

    plsc.subcore_barrier()
    pltpu.sync_copy(acc.at[pl.ds(s * RPT, RPT)],
                    out_hbm.at[c, pl.ds(s * RPT, RPT)])


@functools.cache
def _make_sc_aggregate(width):
  kern = functools.partial(
      pl.kernel,
      out_type=jax.ShapeDtypeStruct((NC, NPAD, width), jnp.float32),
      mesh=plsc.VectorSubcoreMesh(core_axis_name="c", subcore_axis_name="s"),
      scratch_types=(
          [pltpu.VMEM((BMAX, BLK), jnp.int32),
           pltpu.VMEM((BMAX, BLK), jnp.int32)]
          + [pltpu.VMEM((BLK, width), jnp.float32) for _ in range(NBUF)]
          + [pltpu.VMEM_SHARED((NPAD, width), jnp.float32)]
          + [pltpu.SemaphoreType.DMA for _ in range(NBUF)]
      ),
      compiler_params=pltpu.CompilerParams(use_tc_tiling_on_sc=False),
  )
  return kern(functools.partial(_sc_aggregate_body, width))


def _sc_aggregate_body(width, g_hbm, src_hbm, dst_hbm, zeros_hbm, out_hbm,
                       srcv, dstv, *rest):
    bufs = rest[:NBUF]
    acc = rest[NBUF]
    sems = rest[NBUF + 1:]
    c = lax.axis_index("c")
    s = lax.axis_index("s")
    row0 = _tile_rows(c, s)

    pltpu.sync_copy(zeros_hbm, acc.at[pl.ds(s * RPT, RPT)])
    plsc.subcore_barrier()

    def run(nblk):
        pltpu.sync_copy(src_hbm.at[pl.ds(row0, nblk)],
                        srcv.at[pl.ds(0, nblk)])
        pltpu.sync_copy(dst_hbm.at[pl.ds(row0, nblk)],
                        dstv.at[pl.ds(0, nblk)])

        def ring(i, _):
            for k in range(NBUF):
                j = i * NBUF + k

                @pl.when(i > 0)
                def _():
                    pltpu.make_async_copy(bufs[k],
                                          acc.at[dstv.at[j - NBUF]],
                                          sems[k]).wait()
                pltpu.async_copy(g_hbm.at[srcv.at[j]], bufs[k], sems[k])
            for k in range(NBUF):
                j = i * NBUF + k
                pltpu.make_async_copy(g_hbm.at[srcv.at[j]], bufs[k],
                                      sems[k]).wait()
                pltpu.async_copy(bufs[k], acc.at[dstv.at[j]], sems[k],
                                 add=True)
            return 0

        lax.fori_loop(0, nblk // NBUF, ring, 0)
        for k in range(NBUF):
            pltpu.make_async_copy(bufs[k], acc.at[dstv.at[nblk - NBUF + k]],
                                  sems[k]).wait()

    @pl.when(c == 0)
    def _():
        run(B0)

    @pl.when(c == 1)
    def _():
        run(B1)

    plsc.subcore_barrier()
    pltpu.sync_copy(acc.at[pl.ds(s * RPT, RPT)],
                    out_hbm.at[c, pl.ds(s * RPT, RPT)])


def _tc_mm_prep_body(x_ref, w_ref, deg_ref, g_ref, dinv_ref):
    h = jnp.dot(x_ref[...], w_ref[...], preferred_element_type=jnp.float32)
    deg = deg_ref[0] + deg_ref[1] + 1.0
    dinv = lax.rsqrt(deg)
    dinv_ref[...] = dinv
    g_ref[...] = h * dinv


def _tc_layer2_body(q_ref, g1_ref, dinv_ref, b1_ref, w2_ref, g2_ref):
    dinv = dinv_ref[...]
    z = (q_ref[0] + q_ref[1] + g1_ref[...]) * dinv + b1_ref[...]
    z = jnp.maximum(z, 0.0)
    h2 = jnp.dot(z, w2_ref[...], preferred_element_type=jnp.float32)
    g2_ref[...] = h2 * dinv


def _tc_final_body(r_ref, g2_ref, dinv_ref, b2_ref, o_ref):
    logits = (r_ref[0] + r_ref[1] + g2_ref[...]) * dinv_ref[...] + b2_ref[...]
    mask = lax.broadcasted_iota(jnp.int32, logits.shape, 1) < 7
    lm = jnp.where(mask, logits, -jnp.inf)
    mx = jnp.max(lm, axis=1, keepdims=True)
    ex = jnp.where(mask, jnp.exp(logits - mx), 0.0)
    lse = jnp.log(jnp.sum(ex, axis=1, keepdims=True))
    o_ref[...] = logits - mx - lse


_ROWS = 1024
_GRID = NPAD // _ROWS

_tc_mm_prep = pl.pallas_call(
    _tc_mm_prep_body,
    grid=(_GRID,),
    in_specs=[pl.BlockSpec((_ROWS, IN_DIM), lambda i: (i, 0)),
              pl.BlockSpec((IN_DIM, HID), lambda i: (0, 0)),
              pl.BlockSpec((NC, _ROWS, 1), lambda i: (0, i, 0))],
    out_specs=[pl.BlockSpec((_ROWS, HID), lambda i: (i, 0)),
               pl.BlockSpec((_ROWS, 1), lambda i: (i, 0))],
    out_shape=[jax.ShapeDtypeStruct((NPAD, HID), jnp.float32),
               jax.ShapeDtypeStruct((NPAD, 1), jnp.float32)],
)

_tc_layer2 = pl.pallas_call(
    _tc_layer2_body,
    grid=(_GRID,),
    in_specs=[pl.BlockSpec((NC, _ROWS, HID), lambda i: (0, i, 0)),
              pl.BlockSpec((_ROWS, HID), lambda i: (i, 0)),
              pl.BlockSpec((_ROWS, 1), lambda i: (i, 0)),
              pl.BlockSpec((1, HID), lambda i: (0, 0)),
              pl.BlockSpec((HID, OUT2), lambda i: (0, 0))],
    out_specs=pl.BlockSpec((_ROWS, OUT2), lambda i: (i, 0)),
    out_shape=jax.ShapeDtypeStruct((NPAD, OUT2), jnp.float32),
)

_tc_final = pl.pallas_call(
    _tc_final_body,
    grid=(_GRID,),
    in_specs=[pl.BlockSpec((NC, _ROWS, OUT2), lambda i: (0, i, 0)),
              pl.BlockSpec((_ROWS, OUT2), lambda i: (i, 0)),
              pl.BlockSpec((_ROWS, 1), lambda i: (i, 0)),
              pl.BlockSpec((1, OUT2), lambda i: (0, 0))],
    out_specs=pl.BlockSpec((_ROWS, OUT2), lambda i: (i, 0)),
    out_shape=jax.ShapeDtypeStruct((NPAD, OUT2), jnp.float32),
)


@jax.jit
def kernel(x, edge_index, W1, b1, W2, b2):
    ei = edge_index.astype(jnp.int32)
    pad = jnp.full((EPAD - E,), N, jnp.int32)
    src2 = jnp.concatenate([ei[0], pad]).reshape(EB, BLK)
    dst2 = jnp.concatenate([ei[1], pad]).reshape(EB, BLK)
    xp = jnp.pad(x, ((0, NPAD - N), (0, 0)))
    w2p = jnp.zeros((HID, OUT2), jnp.float32).at[:, :7].set(W2)
    b1r = b1.reshape(1, HID)
    b2r = jnp.zeros((1, OUT2), jnp.float32).at[0, :7].set(b2)
    z16 = jnp.zeros((RPT, HID), jnp.float32)
    z8 = jnp.zeros((RPT, OUT2), jnp.float32)

    deg_parts = _make_sc_degree()(dst2).reshape(NC, NPAD, 1)
    g1, dinv = _tc_mm_prep(xp, W1, deg_parts)
    q = _make_sc_aggregate(HID)(g1, src2, dst2, z16)
    g2 = _tc_layer2(q, g1, dinv, b1r, w2p)
    r = _make_sc_aggregate(OUT2)(g2, src2, dst2, z8)
    out = _tc_final(r, g2, dinv, b2r)
    return out[:N, :7]


# speedup vs baseline: 1.2153x; 1.0853x over previous
import functools

import jax
import jax.numpy as jnp
from jax import lax
from jax.experimental import pallas as pl
from jax.experimental.pallas import tpu as pltpu
from jax.experimental.pallas import tpu_sc as plsc

N = 10000
E = 320000
IN_DIM = 128
HID = 16
OUT2 = 8

NC = 2
NS = 16
BLK = 128
EB = 2560
EPAD = EB * BLK
B0 = 108
B1 = 52
BMAX = max(B0, B1)
NPAD = 10240
RPT = NPAD // NS
NBUF = 4


def _tile_rows(c, s):
    row0 = jnp.where(c == 0, s * B0, NS * B0 + s * B1)
    return row0


@functools.cache
def _make_sc_degree():
  kern = functools.partial(
      pl.kernel,
      out_type=jax.ShapeDtypeStruct((NC, NPAD, HID), jnp.float32),
      mesh=plsc.VectorSubcoreMesh(core_axis_name="c", subcore_axis_name="s"),
      scratch_types=[
          pltpu.VMEM((BMAX, BLK), jnp.int32),
          pltpu.VMEM((BLK,), jnp.float32),
          pltpu.VMEM((BLK,), jnp.float32),
          pltpu.VMEM((RPT,), jnp.float32),
          pltpu.VMEM((RPT, HID), jnp.float32),
          pltpu.VMEM_SHARED((NPAD,), jnp.float32),
          pltpu.SemaphoreType.DMA,
      ],
      compiler_params=pltpu.CompilerParams(use_tc_tiling_on_sc=False,
                                           needs_layout_passes=False),
  )
  return kern(_sc_degree_body)


def _sc_degree_body(dst_hbm, out_hbm, dstv, ones_v, zbuf, degv, e16, acc,
                    sem):
    c = lax.axis_index("c")
    s = lax.axis_index("s")
    row0 = _tile_rows(c, s)

    def fill(i, _):
        zbuf[pl.ds(i * 16, 16)] = jnp.zeros((16,), jnp.float32)
        ones_v[pl.ds(i * 16, 16)] = jnp.ones((16,), jnp.float32)
        return 0

    lax.fori_loop(0, BLK // 16, fill, 0)

    def zero(i, _):
        pltpu.sync_copy(zbuf, acc.at[pl.ds(s * RPT + i * BLK, BLK)])
        return 0

    lax.fori_loop(0, RPT // BLK, zero, 0)
    plsc.subcore_barrier()

    def run(nblk):
        pltpu.sync_copy(dst_hbm.at[pl.ds(row0, nblk)],
                        dstv.at[pl.ds(0, nblk)])

        def blk(j, _):
            pltpu.async_copy(ones_v, acc.at[dstv.at[j]], sem, add=True)
            return 0

        lax.fori_loop(0, nblk, blk, 0)

        def drain(j, _):
            pltpu.make_async_copy(ones_v, acc.at[dstv.at[j]], sem).wait()
            return 0

        lax.fori_loop(0, nblk, drain, 0)

    @pl.when(c == 0)
    def _():
        run(B0)

    @pl.when(c == 1)
    def _():
        run(B1)

    plsc.subcore_barrier()
    pltpu.sync_copy(acc.at[pl.ds(s * RPT, RPT)], degv)

    def expand(i, _):
        idx = jnp.full((16,), i, jnp.int32)
        e16[i] = plsc.load_gather(degv, [idx])
        return 0

    lax.fori_loop(0, RPT, expand, 0)
    pltpu.sync_copy(e16, out_hbm.at[c, pl.ds(s * RPT, RPT)])


@functools.cache
def _make_sc_aggregate(width):
  kern = functools.partial(
      pl.kernel,
      out_type=jax.ShapeDtypeStruct((NC, NPAD, width), jnp.float32),
      mesh=plsc.VectorSubcoreMesh(core_axis_name="c", subcore_axis_name="s"),
      scratch_types=(
          [pltpu.VMEM((BMAX, BLK), jnp.int32),
           pltpu.VMEM((BMAX, BLK), jnp.int32)]
          + [pltpu.VMEM((BLK, width), jnp.float32) for _ in range(NBUF)]
          + [pltpu.VMEM_SHARED((NPAD, width), jnp.float32)]
          + [pltpu.SemaphoreType.DMA for _ in range(NBUF)]
      ),
      compiler_params=pltpu.CompilerParams(use_tc_tiling_on_sc=False),
  )
  return kern(functools.partial(_sc_aggregate_body, width))


def _sc_aggregate_body(width, g_hbm, src_hbm, dst_hbm, zeros_hbm, out_hbm,
                       srcv, dstv, *rest):
    bufs = rest[:NBUF]
    acc = rest[NBUF]
    sems = rest[NBUF + 1:]
    c = lax.axis_index("c")
    s = lax.axis_index("s")
    row0 = _tile_rows(c, s)

    pltpu.sync_copy(zeros_hbm, acc.at[pl.ds(s * RPT, RPT)])
    plsc.subcore_barrier()

    def run(nblk):
        pltpu.sync_copy(src_hbm.at[pl.ds(row0, nblk)],
                        srcv.at[pl.ds(0, nblk)])
        pltpu.sync_copy(dst_hbm.at[pl.ds(row0, nblk)],
                        dstv.at[pl.ds(0, nblk)])

        def ring(i, _):
            for k in range(NBUF):
                j = i * NBUF + k

                @pl.when(i > 0)
                def _():
                    pltpu.make_async_copy(bufs[k],
                                          acc.at[dstv.at[j - NBUF]],
                                          sems[k]).wait()
                pltpu.async_copy(g_hbm.at[srcv.at[j]], bufs[k], sems[k])
            for k in range(NBUF):
                j = i * NBUF + k
                pltpu.make_async_copy(g_hbm.at[srcv.at[j]], bufs[k],
                                      sems[k]).wait()
                pltpu.async_copy(bufs[k], acc.at[dstv.at[j]], sems[k],
                                 add=True)
            return 0

        lax.fori_loop(0, nblk // NBUF, ring, 0)
        for k in range(NBUF):
            pltpu.make_async_copy(bufs[k], acc.at[dstv.at[nblk - NBUF + k]],
                                  sems[k]).wait()

    @pl.when(c == 0)
    def _():
        run(B0)

    @pl.when(c == 1)
    def _():
        run(B1)

    plsc.subcore_barrier()
    pltpu.sync_copy(acc.at[pl.ds(s * RPT, RPT)],
                    out_hbm.at[c, pl.ds(s * RPT, RPT)])


PR = NPAD // 8
_BR = 128
_GRID = PR // _BR


def _tc_mm_prep_body(xg_ref, wbd_ref, deg_ref, g_ref, dinv_ref):
    h = jnp.dot(xg_ref[...], wbd_ref[...], preferred_element_type=jnp.float32, precision=lax.Precision.HIGHEST)
    dinv = lax.rsqrt(deg_ref[0] + deg_ref[1] + 1.0)
    dinv_ref[...] = dinv
    g_ref[...] = h * dinv


def _tc_layer2_body(q_ref, g1_ref, dinv_ref, b1_ref, w2bd_ref, g2_ref):
    dinv = dinv_ref[...]
    z = (q_ref[0] + q_ref[1] + g1_ref[...]) * dinv + b1_ref[...]
    z = jnp.maximum(z, 0.0)
    h2 = jnp.dot(z, w2bd_ref[...], preferred_element_type=jnp.float32, precision=lax.Precision.HIGHEST)
    g2_ref[...] = h2 * dinv


def _tc_final_body(r_ref, g2_ref, dinv_ref, b2_ref, gsum_ref, o_ref):
    logits = (r_ref[0] + r_ref[1] + g2_ref[...]) * dinv_ref[...] + b2_ref[...]
    mask = lax.broadcasted_iota(jnp.int32, logits.shape, 1) % HID < 7
    lm = jnp.where(mask, logits, -jnp.inf)
    mx = jnp.max(lm, axis=1, keepdims=True)
    ex = jnp.where(mask, jnp.exp(logits - mx), 0.0)
    gs = jnp.dot(ex, gsum_ref[...], preferred_element_type=jnp.float32, precision=lax.Precision.HIGHEST)
    o_ref[...] = logits - mx - jnp.log(gs)


_tc_mm_prep = pl.pallas_call(
    _tc_mm_prep_body,
    grid=(_GRID,),
    in_specs=[pl.BlockSpec((_BR, 8 * IN_DIM), lambda i: (i, 0)),
              pl.BlockSpec((8 * IN_DIM, 128), lambda i: (0, 0)),
              pl.BlockSpec((NC, _BR, 128), lambda i: (0, i, 0))],
    out_specs=[pl.BlockSpec((_BR, 128), lambda i: (i, 0)),
               pl.BlockSpec((_BR, 128), lambda i: (i, 0))],
    out_shape=[jax.ShapeDtypeStruct((PR, 128), jnp.float32),
               jax.ShapeDtypeStruct((PR, 128), jnp.float32)],
)

_tc_layer2 = pl.pallas_call(
    _tc_layer2_body,
    grid=(_GRID,),
    in_specs=[pl.BlockSpec((NC, _BR, 128), lambda i: (0, i, 0)),
              pl.BlockSpec((_BR, 128), lambda i: (i, 0)),
              pl.BlockSpec((_BR, 128), lambda i: (i, 0)),
              pl.BlockSpec((1, 128), lambda i: (0, 0)),
              pl.BlockSpec((128, 128), lambda i: (0, 0))],
    out_specs=pl.BlockSpec((_BR, 128), lambda i: (i, 0)),
    out_shape=jax.ShapeDtypeStruct((PR, 128), jnp.float32),
)

_tc_final = pl.pallas_call(
    _tc_final_body,
    grid=(_GRID,),
    in_specs=[pl.BlockSpec((NC, _BR, 128), lambda i: (0, i, 0)),
              pl.BlockSpec((_BR, 128), lambda i: (i, 0)),
              pl.BlockSpec((_BR, 128), lambda i: (i, 0)),
              pl.BlockSpec((1, 128), lambda i: (0, 0)),
              pl.BlockSpec((128, 128), lambda i: (0, 0))],
    out_specs=pl.BlockSpec((_BR, 128), lambda i: (i, 0)),
    out_shape=jax.ShapeDtypeStruct((PR, 128), jnp.float32),
)


@jax.jit
def kernel(x, edge_index, W1, b1, W2, b2):
    ei = edge_index.astype(jnp.int32)
    pad = jnp.full((EPAD - E,), N, jnp.int32)
    src2 = jnp.concatenate([ei[0], pad]).reshape(EB, BLK)
    dst2 = jnp.concatenate([ei[1], pad]).reshape(EB, BLK)
    xg = jnp.pad(x, ((0, NPAD - N), (0, 0))).reshape(PR, 8 * IN_DIM)
    eye8 = jnp.eye(8, dtype=jnp.float32)
    w1bd = jnp.kron(eye8, W1)
    w2p = jnp.zeros((HID, HID), jnp.float32).at[:, :7].set(W2)
    w2bd = jnp.kron(eye8, w2p)
    gsum = jnp.kron(eye8, jnp.ones((HID, HID), jnp.float32))
    b1t = jnp.tile(b1, 8).reshape(1, 128)
    b2p = jnp.zeros((HID,), jnp.float32).at[:7].set(b2)
    b2t = jnp.tile(b2p, 8).reshape(1, 128)
    z16 = jnp.zeros((RPT, HID), jnp.float32)

    deg16 = _make_sc_degree()(dst2).reshape(NC, PR, 128)
    g1p, dinvp = _tc_mm_prep(xg, w1bd, deg16)
    g1 = g1p.reshape(NPAD, HID)
    q = _make_sc_aggregate(HID)(g1, src2, dst2, z16)
    g2p = _tc_layer2(q.reshape(NC, PR, 128), g1p, dinvp, b1t, w2bd)
    r = _make_sc_aggregate(HID)(g2p.reshape(NPAD, HID), src2, dst2, z16)
    outp = _tc_final(r.reshape(NC, PR, 128), g2p, dinvp, b2t, gsum)
    return outp.reshape(NPAD, HID)[:N, :7]


# speedup vs baseline: 1.2554x; 1.0330x over previous
import functools

import jax
import jax.numpy as jnp
from jax import lax
from jax.experimental import pallas as pl
from jax.experimental.pallas import tpu as pltpu
from jax.experimental.pallas import tpu_sc as plsc

N = 10000
E = 320000
IN_DIM = 128
HID = 16
OUT2 = 8

NC = 2
NS = 16
BLK = 128
EB = 2560
EPAD = EB * BLK
B0 = 120
B1 = 40
BMAX = max(B0, B1)
NPAD = 10240
RPT = NPAD // NS
NBUF = 4


def _tile_rows(c, s):
    row0 = jnp.where(c == 0, s * B0, NS * B0 + s * B1)
    return row0


@functools.cache
def _make_sc_degree():
  kern = functools.partial(
      pl.kernel,
      out_type=jax.ShapeDtypeStruct((NC, NPAD, HID), jnp.float32),
      mesh=plsc.VectorSubcoreMesh(core_axis_name="c", subcore_axis_name="s"),
      scratch_types=[
          pltpu.VMEM((BMAX, BLK), jnp.int32),
          pltpu.VMEM((BLK,), jnp.float32),
          pltpu.VMEM((BLK,), jnp.float32),
          pltpu.VMEM((RPT,), jnp.float32),
          pltpu.VMEM((RPT, HID), jnp.float32),
          pltpu.VMEM_SHARED((NPAD,), jnp.float32),
          pltpu.SemaphoreType.DMA,
      ],
      compiler_params=pltpu.CompilerParams(use_tc_tiling_on_sc=False,
                                           needs_layout_passes=False),
  )
  return kern(_sc_degree_body)


def _sc_degree_body(dst_hbm, out_hbm, dstv, ones_v, zbuf, degv, e16, acc,
                    sem):
    c = lax.axis_index("c")
    s = lax.axis_index("s")
    row0 = _tile_rows(c, s)

    def fill(i, _):
        zbuf[pl.ds(i * 16, 16)] = jnp.zeros((16,), jnp.float32)
        ones_v[pl.ds(i * 16, 16)] = jnp.ones((16,), jnp.float32)
        return 0

    lax.fori_loop(0, BLK // 16, fill, 0)

    def zero(i, _):
        pltpu.sync_copy(zbuf, acc.at[pl.ds(s * RPT + i * BLK, BLK)])
        return 0

    lax.fori_loop(0, RPT // BLK, zero, 0)
    plsc.subcore_barrier()

    def run(nblk):
        pltpu.sync_copy(dst_hbm.at[pl.ds(row0, nblk)],
                        dstv.at[pl.ds(0, nblk)])

        def blk(j, _):
            pltpu.async_copy(ones_v, acc.at[dstv.at[j]], sem, add=True)
            return 0

        lax.fori_loop(0, nblk, blk, 0)

        def drain(j, _):
            pltpu.make_async_copy(ones_v, acc.at[dstv.at[j]], sem).wait()
            return 0

        lax.fori_loop(0, nblk, drain, 0)

    @pl.when(c == 0)
    def _():
        run(B0)

    @pl.when(c == 1)
    def _():
        run(B1)

    plsc.subcore_barrier()
    pltpu.sync_copy(acc.at[pl.ds(s * RPT, RPT)], degv)

    def expand(i, _):
        idx = jnp.full((16,), i, jnp.int32)
        e16[i] = plsc.load_gather(degv, [idx])
        return 0

    lax.fori_loop(0, RPT, expand, 0)
    pltpu.sync_copy(e16, out_hbm.at[c, pl.ds(s * RPT, RPT)])


@functools.cache
def _make_sc_aggregate(width):
  kern = functools.partial(
      pl.kernel,
      out_type=jax.ShapeDtypeStruct((NC, NPAD, width), jnp.float32),
      mesh=plsc.VectorSubcoreMesh(core_axis_name="c", subcore_axis_name="s"),
      scratch_types=(
          [pltpu.VMEM((BMAX, BLK), jnp.int32),
           pltpu.VMEM((BMAX, BLK), jnp.int32)]
          + [pltpu.VMEM((BLK, width), jnp.float32) for _ in range(NBUF)]
          + [pltpu.VMEM_SHARED((NPAD, width), jnp.float32)]
          + [pltpu.SemaphoreType.DMA for _ in range(NBUF)]
      ),
      compiler_params=pltpu.CompilerParams(use_tc_tiling_on_sc=False),
  )
  return kern(functools.partial(_sc_aggregate_body, width))


def _sc_aggregate_body(width, g_hbm, src_hbm, dst_hbm, zeros_hbm, out_hbm,
                       srcv, dstv, *rest):
    bufs = rest[:NBUF]
    acc = rest[NBUF]
    sems = rest[NBUF + 1:]
    c = lax.axis_index("c")
    s = lax.axis_index("s")
    row0 = _tile_rows(c, s)

    pltpu.sync_copy(zeros_hbm, acc.at[pl.ds(s * RPT, RPT)])
    plsc.subcore_barrier()

    def run(nblk):
        pltpu.sync_copy(src_hbm.at[pl.ds(row0, nblk)],
                        srcv.at[pl.ds(0, nblk)])
        pltpu.sync_copy(dst_hbm.at[pl.ds(row0, nblk)],
                        dstv.at[pl.ds(0, nblk)])

        def ring(i, _):
            for k in range(NBUF):
                j = i * NBUF + k

                @pl.when(i > 0)
                def _():
                    pltpu.make_async_copy(bufs[k],
                                          acc.at[dstv.at[j - NBUF]],
                                          sems[k]).wait()
                pltpu.async_copy(g_hbm.at[srcv.at[j]], bufs[k], sems[k])
            for k in range(NBUF):
                j = i * NBUF + k
                pltpu.make_async_copy(g_hbm.at[srcv.at[j]], bufs[k],
                                      sems[k]).wait()
                pltpu.async_copy(bufs[k], acc.at[dstv.at[j]], sems[k],
                                 add=True)
            return 0

        lax.fori_loop(0, nblk // NBUF, ring, 0)
        for k in range(NBUF):
            pltpu.make_async_copy(bufs[k], acc.at[dstv.at[nblk - NBUF + k]],
                                  sems[k]).wait()

    @pl.when(c == 0)
    def _():
        run(B0)

    @pl.when(c == 1)
    def _():
        run(B1)

    plsc.subcore_barrier()
    pltpu.sync_copy(acc.at[pl.ds(s * RPT, RPT)],
                    out_hbm.at[c, pl.ds(s * RPT, RPT)])


PR = NPAD // 8
_BR = 128
_GRID = PR // _BR


def _tc_mm_prep_body(xg_ref, wbd_ref, deg_ref, g_ref, dinv_ref):
    h = jnp.dot(xg_ref[...], wbd_ref[...], preferred_element_type=jnp.float32, precision=lax.Precision.HIGHEST)
    dinv = lax.rsqrt(deg_ref[0] + deg_ref[1] + 1.0)
    dinv_ref[...] = dinv
    g_ref[...] = h * dinv


def _tc_layer2_body(q_ref, g1_ref, dinv_ref, b1_ref, w2bd_ref, g2_ref):
    dinv = dinv_ref[...]
    z = (q_ref[0] + q_ref[1] + g1_ref[...]) * dinv + b1_ref[...]
    z = jnp.maximum(z, 0.0)
    h2 = jnp.dot(z, w2bd_ref[...], preferred_element_type=jnp.float32, precision=lax.Precision.HIGHEST)
    g2_ref[...] = h2 * dinv


def _tc_final_body(r_ref, g2_ref, dinv_ref, b2_ref, gsum_ref, o_ref):
    logits = (r_ref[0] + r_ref[1] + g2_ref[...]) * dinv_ref[...] + b2_ref[...]
    mask = lax.broadcasted_iota(jnp.int32, logits.shape, 1) % HID < 7
    lm = jnp.where(mask, logits, -jnp.inf)
    mx = jnp.max(lm, axis=1, keepdims=True)
    ex = jnp.where(mask, jnp.exp(logits - mx), 0.0)
    gs = jnp.dot(ex, gsum_ref[...], preferred_element_type=jnp.float32, precision=lax.Precision.HIGHEST)
    o_ref[...] = logits - mx - jnp.log(gs)


_tc_mm_prep = pl.pallas_call(
    _tc_mm_prep_body,
    grid=(_GRID,),
    in_specs=[pl.BlockSpec((_BR, 8 * IN_DIM), lambda i: (i, 0)),
              pl.BlockSpec((8 * IN_DIM, 128), lambda i: (0, 0)),
              pl.BlockSpec((NC, _BR, 128), lambda i: (0, i, 0))],
    out_specs=[pl.BlockSpec((_BR, 128), lambda i: (i, 0)),
               pl.BlockSpec((_BR, 128), lambda i: (i, 0))],
    out_shape=[jax.ShapeDtypeStruct((PR, 128), jnp.float32),
               jax.ShapeDtypeStruct((PR, 128), jnp.float32)],
)

_tc_layer2 = pl.pallas_call(
    _tc_layer2_body,
    grid=(_GRID,),
    in_specs=[pl.BlockSpec((NC, _BR, 128), lambda i: (0, i, 0)),
              pl.BlockSpec((_BR, 128), lambda i: (i, 0)),
              pl.BlockSpec((_BR, 128), lambda i: (i, 0)),
              pl.BlockSpec((1, 128), lambda i: (0, 0)),
              pl.BlockSpec((128, 128), lambda i: (0, 0))],
    out_specs=pl.BlockSpec((_BR, 128), lambda i: (i, 0)),
    out_shape=jax.ShapeDtypeStruct((PR, 128), jnp.float32),
)

_tc_final = pl.pallas_call(
    _tc_final_body,
    grid=(_GRID,),
    in_specs=[pl.BlockSpec((NC, _BR, 128), lambda i: (0, i, 0)),
              pl.BlockSpec((_BR, 128), lambda i: (i, 0)),
              pl.BlockSpec((_BR, 128), lambda i: (i, 0)),
              pl.BlockSpec((1, 128), lambda i: (0, 0)),
              pl.BlockSpec((128, 128), lambda i: (0, 0))],
    out_specs=pl.BlockSpec((_BR, 128), lambda i: (i, 0)),
    out_shape=jax.ShapeDtypeStruct((PR, 128), jnp.float32),
)


@jax.jit
def kernel(x, edge_index, W1, b1, W2, b2):
    ei = edge_index.astype(jnp.int32)
    pad = jnp.full((EPAD - E,), N, jnp.int32)
    src2 = jnp.concatenate([ei[0], pad]).reshape(EB, BLK)
    dst2 = jnp.concatenate([ei[1], pad]).reshape(EB, BLK)
    xg = jnp.pad(x, ((0, NPAD - N), (0, 0))).reshape(PR, 8 * IN_DIM)
    eye8 = jnp.eye(8, dtype=jnp.float32)
    w1bd = jnp.kron(eye8, W1)
    w2p = jnp.zeros((HID, HID), jnp.float32).at[:, :7].set(W2)
    w2bd = jnp.kron(eye8, w2p)
    gsum = jnp.kron(eye8, jnp.ones((HID, HID), jnp.float32))
    b1t = jnp.tile(b1, 8).reshape(1, 128)
    b2p = jnp.zeros((HID,), jnp.float32).at[:7].set(b2)
    b2t = jnp.tile(b2p, 8).reshape(1, 128)
    z16 = jnp.zeros((RPT, HID), jnp.float32)

    deg16 = _make_sc_degree()(dst2).reshape(NC, PR, 128)
    g1p, dinvp = _tc_mm_prep(xg, w1bd, deg16)
    g1 = g1p.reshape(NPAD, HID)
    q = _make_sc_aggregate(HID)(g1, src2, dst2, z16)
    g2p = _tc_layer2(q.reshape(NC, PR, 128), g1p, dinvp, b1t, w2bd)
    r = _make_sc_aggregate(HID)(g2p.reshape(NPAD, HID), src2, dst2, z16)
    outp = _tc_final(r.reshape(NC, PR, 128), g2p, dinvp, b2t, gsum)
    return outp.reshape(NPAD, HID)[:N, :7]


# speedup vs baseline: 1.5685x; 1.2495x over previous
import functools

import jax
import jax.numpy as jnp
from jax import lax
from jax.experimental import pallas as pl
from jax.experimental.pallas import tpu as pltpu
from jax.experimental.pallas import tpu_sc as plsc

N = 10000
E = 320000
IN_DIM = 128
HID = 16
OUT2 = 8

NC = 2
NS = 16
BLK = 128
EB = E // BLK
B0 = 120
B1 = 36
BMAX = max(B0, B1 + 1)
NPAD = 10240
RPT = NPAD // NS
NBUF = 4


def _tile_rows(c, s):
    row0 = jnp.where(c == 0, s * B0, NS * B0 + s * B1)
    return row0


@functools.cache
def _make_sc_degree():
  kern = functools.partial(
      pl.kernel,
      out_type=jax.ShapeDtypeStruct((NC, NPAD, HID), jnp.float32),
      mesh=plsc.VectorSubcoreMesh(core_axis_name="c", subcore_axis_name="s"),
      scratch_types=[
          pltpu.VMEM((BMAX, BLK), jnp.int32),
          pltpu.VMEM((BLK,), jnp.float32),
          pltpu.VMEM((BLK,), jnp.float32),
          pltpu.VMEM((RPT,), jnp.float32),
          pltpu.VMEM((RPT, HID), jnp.float32),
          pltpu.VMEM_SHARED((NPAD,), jnp.float32),
          pltpu.SemaphoreType.DMA,
      ],
      compiler_params=pltpu.CompilerParams(use_tc_tiling_on_sc=False,
                                           needs_layout_passes=False),
  )
  return kern(_sc_degree_body)


def _sc_degree_body(dst_hbm, out_hbm, dstv, ones_v, zbuf, degv, e16, acc,
                    sem):
    c = lax.axis_index("c")
    s = lax.axis_index("s")
    row0 = _tile_rows(c, s)

    def fill(i, _):
        zbuf[pl.ds(i * 16, 16)] = jnp.zeros((16,), jnp.float32)
        ones_v[pl.ds(i * 16, 16)] = jnp.ones((16,), jnp.float32)
        return 0

    lax.fori_loop(0, BLK // 16, fill, 0)

    def zero(i, _):
        pltpu.sync_copy(zbuf, acc.at[pl.ds(s * RPT + i * BLK, BLK)])
        return 0

    lax.fori_loop(0, RPT // BLK, zero, 0)
    plsc.subcore_barrier()

    extra = jnp.logical_and(c == 1, s < 4)

    def run(nblk):
        pltpu.sync_copy(dst_hbm.at[pl.ds(row0, nblk)],
                        dstv.at[pl.ds(0, nblk)])
        if nblk == B1:
            @pl.when(extra)
            def _():
                pltpu.sync_copy(dst_hbm.at[NS * B0 + NS * B1 + s],
                                dstv.at[B1])
        nrun = nblk + extra.astype(jnp.int32) if nblk == B1 else nblk

        def blk(j, _):
            pltpu.async_copy(ones_v, acc.at[dstv.at[j]], sem, add=True)
            return 0

        lax.fori_loop(0, nrun, blk, 0)

        def drain(j, _):
            pltpu.make_async_copy(ones_v, acc.at[dstv.at[j]], sem).wait()
            return 0

        lax.fori_loop(0, nrun, drain, 0)

    @pl.when(c == 0)
    def _():
        run(B0)

    @pl.when(c == 1)
    def _():
        run(B1)

    plsc.subcore_barrier()
    pltpu.sync_copy(acc.at[pl.ds(s * RPT, RPT)], degv)

    def expand(i, _):
        idx = jnp.full((16,), i, jnp.int32)
        e16[i] = plsc.load_gather(degv, [idx])
        return 0

    lax.fori_loop(0, RPT, expand, 0)
    pltpu.sync_copy(e16, out_hbm.at[c, pl.ds(s * RPT, RPT)])


@functools.cache
def _make_sc_aggregate(width):
  kern = functools.partial(
      pl.kernel,
      out_type=jax.ShapeDtypeStruct((NC, NPAD, width), jnp.float32),
      mesh=plsc.VectorSubcoreMesh(core_axis_name="c", subcore_axis_name="s"),
      scratch_types=(
          [pltpu.VMEM((BMAX, BLK), jnp.int32),
           pltpu.VMEM((BMAX, BLK), jnp.int32)]
          + [pltpu.VMEM((BLK, width), jnp.float32) for _ in range(NBUF)]
          + [pltpu.VMEM_SHARED((NPAD, width), jnp.float32)]
          + [pltpu.SemaphoreType.DMA for _ in range(NBUF)]
      ),
      compiler_params=pltpu.CompilerParams(use_tc_tiling_on_sc=False),
  )
  return kern(functools.partial(_sc_aggregate_body, width))


def _sc_aggregate_body(width, g_hbm, src_hbm, dst_hbm, zeros_hbm, out_hbm,
                       srcv, dstv, *rest):
    bufs = rest[:NBUF]
    acc = rest[NBUF]
    sems = rest[NBUF + 1:]
    c = lax.axis_index("c")
    s = lax.axis_index("s")
    row0 = _tile_rows(c, s)

    pltpu.sync_copy(zeros_hbm, acc.at[pl.ds(s * RPT, RPT)])
    plsc.subcore_barrier()

    extra = jnp.logical_and(c == 1, s < 4)

    def run(nblk):
        pltpu.sync_copy(src_hbm.at[pl.ds(row0, nblk)],
                        srcv.at[pl.ds(0, nblk)])
        pltpu.sync_copy(dst_hbm.at[pl.ds(row0, nblk)],
                        dstv.at[pl.ds(0, nblk)])
        if nblk == B1:
            @pl.when(extra)
            def _():
                pltpu.sync_copy(src_hbm.at[NS * B0 + NS * B1 + s],
                                srcv.at[B1])
                pltpu.sync_copy(dst_hbm.at[NS * B0 + NS * B1 + s],
                                dstv.at[B1])

        def ring(i, _):
            for k in range(NBUF):
                j = i * NBUF + k

                @pl.when(i > 0)
                def _():
                    pltpu.make_async_copy(bufs[k],
                                          acc.at[dstv.at[j - NBUF]],
                                          sems[k]).wait()
                pltpu.async_copy(g_hbm.at[srcv.at[j]], bufs[k], sems[k])
            for k in range(NBUF):
                j = i * NBUF + k
                pltpu.make_async_copy(g_hbm.at[srcv.at[j]], bufs[k],
                                      sems[k]).wait()
                pltpu.async_copy(bufs[k], acc.at[dstv.at[j]], sems[k],
                                 add=True)
            return 0

        lax.fori_loop(0, nblk // NBUF, ring, 0)
        for k in range(NBUF):
            pltpu.make_async_copy(bufs[k], acc.at[dstv.at[nblk - NBUF + k]],
                                  sems[k]).wait()
        if nblk == B1:
            @pl.when(extra)
            def _():
                pltpu.async_copy(g_hbm.at[srcv.at[B1]], bufs[0], sems[0])
                pltpu.make_async_copy(g_hbm.at[srcv.at[B1]], bufs[0],
                                      sems[0]).wait()
                pltpu.async_copy(bufs[0], acc.at[dstv.at[B1]], sems[0],
                                 add=True)
                pltpu.make_async_copy(bufs[0], acc.at[dstv.at[B1]],
                                      sems[0]).wait()

    @pl.when(c == 0)
    def _():
        run(B0)

    @pl.when(c == 1)
    def _():
        run(B1)

    plsc.subcore_barrier()
    pltpu.sync_copy(acc.at[pl.ds(s * RPT, RPT)],
                    out_hbm.at[c, pl.ds(s * RPT, RPT)])


PR = NPAD // 8
_BR = 128
_GRID = PR // _BR


def _tc_mm_prep_body(xg_ref, wbd_ref, deg_ref, g_ref, dinv_ref):
    h = jnp.dot(xg_ref[...], wbd_ref[...], preferred_element_type=jnp.float32, precision=lax.Precision.HIGHEST)
    dinv = lax.rsqrt(deg_ref[0] + deg_ref[1] + 1.0)
    dinv_ref[...] = dinv
    g_ref[...] = h * dinv


def _tc_layer2_body(q_ref, g1_ref, dinv_ref, b1_ref, w2bd_ref, g2_ref):
    dinv = dinv_ref[...]
    z = (q_ref[0] + q_ref[1] + g1_ref[...]) * dinv + b1_ref[...]
    z = jnp.maximum(z, 0.0)
    h2 = jnp.dot(z, w2bd_ref[...], preferred_element_type=jnp.float32, precision=lax.Precision.HIGHEST)
    g2_ref[...] = h2 * dinv


def _tc_final_body(r_ref, g2_ref, dinv_ref, b2_ref, gsum_ref, o_ref):
    logits = (r_ref[0] + r_ref[1] + g2_ref[...]) * dinv_ref[...] + b2_ref[...]
    mask = lax.broadcasted_iota(jnp.int32, logits.shape, 1) % HID < 7
    lm = jnp.where(mask, logits, -jnp.inf)
    mx = jnp.max(lm, axis=1, keepdims=True)
    ex = jnp.where(mask, jnp.exp(logits - mx), 0.0)
    gs = jnp.dot(ex, gsum_ref[...], preferred_element_type=jnp.float32, precision=lax.Precision.HIGHEST)
    o_ref[...] = logits - mx - jnp.log(gs)


_tc_mm_prep = pl.pallas_call(
    _tc_mm_prep_body,
    grid=(_GRID,),
    in_specs=[pl.BlockSpec((_BR, 8 * IN_DIM), lambda i: (i, 0)),
              pl.BlockSpec((8 * IN_DIM, 128), lambda i: (0, 0)),
              pl.BlockSpec((NC, _BR, 128), lambda i: (0, i, 0))],
    out_specs=[pl.BlockSpec((_BR, 128), lambda i: (i, 0)),
               pl.BlockSpec((_BR, 128), lambda i: (i, 0))],
    out_shape=[jax.ShapeDtypeStruct((PR, 128), jnp.float32),
               jax.ShapeDtypeStruct((PR, 128), jnp.float32)],
)

_tc_layer2 = pl.pallas_call(
    _tc_layer2_body,
    grid=(_GRID,),
    in_specs=[pl.BlockSpec((NC, _BR, 128), lambda i: (0, i, 0)),
              pl.BlockSpec((_BR, 128), lambda i: (i, 0)),
              pl.BlockSpec((_BR, 128), lambda i: (i, 0)),
              pl.BlockSpec((1, 128), lambda i: (0, 0)),
              pl.BlockSpec((128, 128), lambda i: (0, 0))],
    out_specs=pl.BlockSpec((_BR, 128), lambda i: (i, 0)),
    out_shape=jax.ShapeDtypeStruct((PR, 128), jnp.float32),
)

_tc_final = pl.pallas_call(
    _tc_final_body,
    grid=(_GRID,),
    in_specs=[pl.BlockSpec((NC, _BR, 128), lambda i: (0, i, 0)),
              pl.BlockSpec((_BR, 128), lambda i: (i, 0)),
              pl.BlockSpec((_BR, 128), lambda i: (i, 0)),
              pl.BlockSpec((1, 128), lambda i: (0, 0)),
              pl.BlockSpec((128, 128), lambda i: (0, 0))],
    out_specs=pl.BlockSpec((_BR, 128), lambda i: (i, 0)),
    out_shape=jax.ShapeDtypeStruct((PR, 128), jnp.float32),
)


@jax.jit
def kernel(x, edge_index, W1, b1, W2, b2):
    ei = edge_index.astype(jnp.int32)
    src2 = ei[0].reshape(EB, BLK)
    dst2 = ei[1].reshape(EB, BLK)
    xg = jnp.pad(x, ((0, NPAD - N), (0, 0))).reshape(PR, 8 * IN_DIM)
    eye8 = jnp.eye(8, dtype=jnp.float32)
    w1bd = jnp.kron(eye8, W1)
    w2p = jnp.zeros((HID, HID), jnp.float32).at[:, :7].set(W2)
    w2bd = jnp.kron(eye8, w2p)
    gsum = jnp.kron(eye8, jnp.ones((HID, HID), jnp.float32))
    b1t = jnp.tile(b1, 8).reshape(1, 128)
    b2p = jnp.zeros((HID,), jnp.float32).at[:7].set(b2)
    b2t = jnp.tile(b2p, 8).reshape(1, 128)
    z16 = jnp.zeros((RPT, HID), jnp.float32)

    deg16 = _make_sc_degree()(dst2).reshape(NC, PR, 128)
    g1p, dinvp = _tc_mm_prep(xg, w1bd, deg16)
    g1 = g1p.reshape(NPAD, HID)
    q = _make_sc_aggregate(HID)(g1, src2, dst2, z16)
    g2p = _tc_layer2(q.reshape(NC, PR, 128), g1p, dinvp, b1t, w2bd)
    r = _make_sc_aggregate(HID)(g2p.reshape(NPAD, HID), src2, dst2, z16)
    outp = _tc_final(r.reshape(NC, PR, 128), g2p, dinvp, b2t, gsum)
    return outp.reshape(NPAD, HID)[:N, :7]
